# baseline (speedup 1.0000x reference)
import jax
import jax.numpy as jnp
from jax.experimental import pallas as pl
from jax.experimental.pallas import tpu as pltpu

_LANE = 128


def _round_up(x, m):
    return ((x + m - 1) // m) * m


def _fused_mlp_logsoftmax(xt_ref, w1_ref, nb1_ref, w2_ref, nb2_ref,
                          w3_ref, b3_ref, ones3_ref, o_ref):
    half = xt_ref.shape[1] // 8
    for k in range(8):
        xk = xt_ref[:, k * half:(k + 1) * half]
        z = jnp.dot(w1_ref[...], xk,
                    preferred_element_type=jnp.float32)
        u = jnp.maximum(z, nb1_ref[...])
        z = jnp.dot(w2_ref[...], u,
                    preferred_element_type=jnp.float32)
        u = jnp.maximum(z, nb2_ref[...])
        lg = jnp.dot(w3_ref[...], u,
                     preferred_element_type=jnp.float32) + b3_ref[...]

        m = jnp.max(lg, axis=0, keepdims=True)
        sh = lg - m
        s = jnp.dot(ones3_ref[...], jnp.exp(sh),
                    preferred_element_type=jnp.float32)
        o_ref[:, k * half:(k + 1) * half] = (sh - jnp.log(s)).astype(o_ref.dtype)


def kernel(x, w1, b1, w2, b2, w3, b3, *, block_batch=131072):
    B, n_in = x.shape
    h1, h2, n_out = w1.shape[0], w2.shape[0], w3.shape[0]

    xt = x.T
    BT = max(_LANE, min(block_batch, _round_up(B, _LANE)))
    Bp = _round_up(B, BT)
    if Bp != B:
        xt = jnp.pad(xt, ((0, 0), (0, Bp - B)))
    grid = (Bp // BT,)

    b2f = w2 @ b1 + b2
    b3f = w3 @ b2f + b3
    ones3 = jnp.ones((1, n_out), jnp.float32)

    flops = 2 * Bp * (n_in * h1 + h1 * h2 + h2 * n_out)
    transcendentals = Bp * (n_out + 1)
    bytes_accessed = (Bp * n_in * 4 + Bp * n_out * 4
                      + (w1.size + w2.size + w3.size
                         + b1.size + b2.size + b3.size) * 4)

    const = lambda i: (0, 0)
    out_t = pl.pallas_call(
        _fused_mlp_logsoftmax,
        out_shape=jax.ShapeDtypeStruct((n_out, Bp), jnp.float32),
        grid=grid,
        in_specs=[
            pl.BlockSpec((n_in, BT), lambda i: (0, i)),
            pl.BlockSpec(w1.shape, const), pl.BlockSpec(b1.shape, const),
            pl.BlockSpec(w2.shape, const), pl.BlockSpec(b2f.shape, const),
            pl.BlockSpec(w3.shape, const), pl.BlockSpec(b3f.shape, const),
            pl.BlockSpec(ones3.shape, const),
        ],
        out_specs=pl.BlockSpec((n_out, BT), lambda i: (0, i)),
        compiler_params=pltpu.CompilerParams(
            dimension_semantics=("parallel",)),
        cost_estimate=pl.CostEstimate(
            flops=flops,
            transcendentals=transcendentals,
            bytes_accessed=bytes_accessed),
    )(xt, w1, -b1, w2, -b2f, w3, b3f, ones3)

    return out_t[:, :B].T


# speedup vs baseline: 1.0120x; 1.0120x over previous
import jax
import jax.numpy as jnp
from jax.experimental import pallas as pl
from jax.experimental.pallas import tpu as pltpu

_LANE = 128


def _round_up(x, m):
    return ((x + m - 1) // m) * m


def _fused_mlp_logsoftmax(xt_ref, w1_ref, nb1_ref, w2_ref, nb2_ref,
                          w3_ref, b3_ref, ones3_ref, o_ref):
    half = xt_ref.shape[1] // 4
    for k in (0, 1, 2, 3):
        xk = xt_ref[:, k * half:(k + 1) * half]
        z = jnp.dot(w1_ref[...], xk,
                    preferred_element_type=jnp.float32)
        u = jnp.maximum(z, nb1_ref[...])
        z = jnp.dot(w2_ref[...], u,
                    preferred_element_type=jnp.float32)
        u = jnp.maximum(z, nb2_ref[...])
        lg = jnp.dot(w3_ref[...], u,
                     preferred_element_type=jnp.float32) + b3_ref[...]

        m = jnp.max(lg, axis=0, keepdims=True)
        sh = lg - m
        s = jnp.dot(ones3_ref[...], jnp.exp(sh),
                    preferred_element_type=jnp.float32)
        o_ref[:, k * half:(k + 1) * half] = (sh - jnp.log(s)).astype(o_ref.dtype)


def kernel(x, w1, b1, w2, b2, w3, b3, *, block_batch=262144):
    B, n_in = x.shape
    h1, h2, n_out = w1.shape[0], w2.shape[0], w3.shape[0]

    xt = x.T
    BT = max(_LANE, min(block_batch, _round_up(B, _LANE)))
    Bp = _round_up(B, BT)
    if Bp != B:
        xt = jnp.pad(xt, ((0, 0), (0, Bp - B)))
    grid = (Bp // BT,)

    b2f = w2 @ b1 + b2
    b3f = w3 @ b2f + b3
    ones3 = jnp.ones((1, n_out), jnp.float32)

    flops = 2 * Bp * (n_in * h1 + h1 * h2 + h2 * n_out)
    transcendentals = Bp * (n_out + 1)
    bytes_accessed = (Bp * n_in * 4 + Bp * n_out * 4
                      + (w1.size + w2.size + w3.size
                         + b1.size + b2.size + b3.size) * 4)

    const = lambda i: (0, 0)
    out_t = pl.pallas_call(
        _fused_mlp_logsoftmax,
        out_shape=jax.ShapeDtypeStruct((n_out, Bp), jnp.float32),
        grid=grid,
        in_specs=[
            pl.BlockSpec((n_in, BT), lambda i: (0, i)),
            pl.BlockSpec(w1.shape, const), pl.BlockSpec(b1.shape, const),
            pl.BlockSpec(w2.shape, const), pl.BlockSpec(b2f.shape, const),
            pl.BlockSpec(w3.shape, const), pl.BlockSpec(b3f.shape, const),
            pl.BlockSpec(ones3.shape, const),
        ],
        out_specs=pl.BlockSpec((n_out, BT), lambda i: (0, i)),
        compiler_params=pltpu.CompilerParams(
            dimension_semantics=("parallel",)),
        cost_estimate=pl.CostEstimate(
            flops=flops,
            transcendentals=transcendentals,
            bytes_accessed=bytes_accessed),
    )(xt, w1, -b1, w2, -b2f, w3, b3f, ones3)

    return out_t[:, :B].T


# speedup vs baseline: 1.0399x; 1.0276x over previous
import jax
import jax.numpy as jnp
from jax.experimental import pallas as pl
from jax.experimental.pallas import tpu as pltpu

_LANE = 128


def _round_up(x, m):
    return ((x + m - 1) // m) * m


def _fused_mlp_logsoftmax(xt_ref, w1_ref, nb1_ref, w2_ref, nb2_ref,
                          w3_ref, b3_ref, ones3_ref, o_ref):
    half = xt_ref.shape[1] // 2
    for k in (0, 1):
        xk = xt_ref[:, k * half:(k + 1) * half]
        z = jnp.dot(w1_ref[...], xk,
                    preferred_element_type=jnp.float32)
        u = jnp.maximum(z, nb1_ref[...])
        z = jnp.dot(w2_ref[...], u,
                    preferred_element_type=jnp.float32)
        u = jnp.maximum(z, nb2_ref[...])
        lg = jnp.dot(w3_ref[...], u,
                     preferred_element_type=jnp.float32) + b3_ref[...]

        m = jnp.max(lg, axis=0, keepdims=True)
        sh = lg - m
        s = jnp.dot(ones3_ref[...], jnp.exp(sh),
                    preferred_element_type=jnp.float32)
        o_ref[:, k * half:(k + 1) * half] = (sh - jnp.log(s)).astype(o_ref.dtype)


def kernel(x, w1, b1, w2, b2, w3, b3, *, block_batch=65536):
    B, n_in = x.shape
    h1, h2, n_out = w1.shape[0], w2.shape[0], w3.shape[0]

    xt = x.T
    BT = max(_LANE, min(block_batch, _round_up(B, _LANE)))
    Bp = _round_up(B, BT)
    if Bp != B:
        xt = jnp.pad(xt, ((0, 0), (0, Bp - B)))
    grid = (Bp // BT,)

    b2f = w2 @ b1 + b2
    b3f = w3 @ b2f + b3
    ones3 = jnp.ones((1, n_out), jnp.float32)

    flops = 2 * Bp * (n_in * h1 + h1 * h2 + h2 * n_out)
    transcendentals = Bp * (n_out + 1)
    bytes_accessed = (Bp * n_in * 4 + Bp * n_out * 4
                      + (w1.size + w2.size + w3.size
                         + b1.size + b2.size + b3.size) * 4)

    const = lambda i: (0, 0)
    out_t = pl.pallas_call(
        _fused_mlp_logsoftmax,
        out_shape=jax.ShapeDtypeStruct((n_out, Bp), jnp.float32),
        grid=grid,
        in_specs=[
            pl.BlockSpec((n_in, BT), lambda i: (0, i)),
            pl.BlockSpec(w1.shape, const), pl.BlockSpec(b1.shape, const),
            pl.BlockSpec(w2.shape, const), pl.BlockSpec(b2f.shape, const),
            pl.BlockSpec(w3.shape, const), pl.BlockSpec(b3f.shape, const),
            pl.BlockSpec(ones3.shape, const),
        ],
        out_specs=pl.BlockSpec((n_out, BT), lambda i: (0, i)),
        compiler_params=pltpu.CompilerParams(
            dimension_semantics=("parallel",)),
        cost_estimate=pl.CostEstimate(
            flops=flops,
            transcendentals=transcendentals,
            bytes_accessed=bytes_accessed),
    )(xt, w1, -b1, w2, -b2f, w3, b3f, ones3)

    return out_t[:, :B].T


# speedup vs baseline: 1.0584x; 1.0178x over previous
import jax
import jax.numpy as jnp
from jax.experimental import pallas as pl
from jax.experimental.pallas import tpu as pltpu

_LANE = 128


def _round_up(x, m):
    return ((x + m - 1) // m) * m


def _fused_mlp_logsoftmax(xt_ref, w1_ref, nb1_ref, w2_ref, nb2_ref,
                          w3_ref, b3_ref, ones3_ref, o_ref):
    half = xt_ref.shape[1] // 4
    for k in (0, 1, 2, 3):
        xk = xt_ref[:, k * half:(k + 1) * half]
        z = jnp.dot(w1_ref[...], xk,
                    preferred_element_type=jnp.float32)
        u = jnp.maximum(z, nb1_ref[...])
        z = jnp.dot(w2_ref[...], u,
                    preferred_element_type=jnp.float32)
        u = jnp.maximum(z, nb2_ref[...])
        lg = jnp.dot(w3_ref[...], u,
                     preferred_element_type=jnp.float32) + b3_ref[...]

        m = jnp.max(lg, axis=0, keepdims=True)
        sh = lg - m
        s = jnp.dot(ones3_ref[...], jnp.exp(sh),
                    preferred_element_type=jnp.float32)
        o_ref[:, k * half:(k + 1) * half] = (sh - jnp.log(s)).astype(o_ref.dtype)


def kernel(x, w1, b1, w2, b2, w3, b3, *, block_batch=131072):
    B, n_in = x.shape
    h1, h2, n_out = w1.shape[0], w2.shape[0], w3.shape[0]

    xt = x.T
    BT = max(_LANE, min(block_batch, _round_up(B, _LANE)))
    Bp = _round_up(B, BT)
    if Bp != B:
        xt = jnp.pad(xt, ((0, 0), (0, Bp - B)))
    grid = (Bp // BT,)

    b2f = w2 @ b1 + b2
    b3f = w3 @ b2f + b3
    ones3 = jnp.ones((1, n_out), jnp.float32)

    flops = 2 * Bp * (n_in * h1 + h1 * h2 + h2 * n_out)
    transcendentals = Bp * (n_out + 1)
    bytes_accessed = (Bp * n_in * 4 + Bp * n_out * 4
                      + (w1.size + w2.size + w3.size
                         + b1.size + b2.size + b3.size) * 4)

    const = lambda i: (0, 0)
    out_t = pl.pallas_call(
        _fused_mlp_logsoftmax,
        out_shape=jax.ShapeDtypeStruct((n_out, Bp), jnp.float32),
        grid=grid,
        in_specs=[
            pl.BlockSpec((n_in, BT), lambda i: (0, i)),
            pl.BlockSpec(w1.shape, const), pl.BlockSpec(b1.shape, const),
            pl.BlockSpec(w2.shape, const), pl.BlockSpec(b2f.shape, const),
            pl.BlockSpec(w3.shape, const), pl.BlockSpec(b3f.shape, const),
            pl.BlockSpec(ones3.shape, const),
        ],
        out_specs=pl.BlockSpec((n_out, BT), lambda i: (0, i)),
        compiler_params=pltpu.CompilerParams(
            dimension_semantics=("parallel",)),
        cost_estimate=pl.CostEstimate(
            flops=flops,
            transcendentals=transcendentals,
            bytes_accessed=bytes_accessed),
    )(xt, w1, -b1, w2, -b2f, w3, b3f, ones3)

    return out_t[:, :B].T
